# baseline (speedup 1.0000x reference)
import functools

import jax
import jax.numpy as jnp
from jax import lax
from jax.experimental import pallas as pl
from jax.experimental.pallas import tpu as pltpu
from jax.experimental.pallas import tpu_sc as plsc

B = 16384
D = 128
NC = 2
NS = 16
NW = NC * NS
TOK_PER_W = B // NW
CH = 128
NCH = TOK_PER_W // CH

_mesh = plsc.VectorSubcoreMesh(core_axis_name="c", subcore_axis_name="s")


@functools.partial(
    pl.kernel,
    out_type=jax.ShapeDtypeStruct((B,), jnp.float32),
    mesh=_mesh,
    compiler_params=pltpu.CompilerParams(needs_layout_passes=False),
    scratch_types=[
        pltpu.VMEM((TOK_PER_W,), jnp.int32),
        pltpu.VMEM((TOK_PER_W,), jnp.int32),
        pltpu.VMEM((TOK_PER_W,), jnp.int32),
        pltpu.VMEM((CH, D), jnp.float32),
        pltpu.VMEM((CH, D), jnp.float32),
        pltpu.VMEM((CH, D), jnp.float32),
        pltpu.VMEM((CH, D), jnp.float32),
        pltpu.VMEM((CH, D), jnp.float32),
        pltpu.VMEM((CH, D), jnp.float32),
        pltpu.VMEM((CH * 16,), jnp.float32),
        pltpu.VMEM((CH,), jnp.float32),
        pltpu.SemaphoreType.DMA,
        pltpu.SemaphoreType.DMA,
    ],
)
def _cp(iu_hbm, ij_hbm, ik_hbm, uemb, iemb, temb, out_hbm,
        idx_u, idx_i, idx_k,
        rows_u0, rows_i0, rows_k0, rows_u1, rows_i1, rows_k1,
        acc_v, out_v, sem0, sem1):
    wid = lax.axis_index("s") * NC + lax.axis_index("c")
    base = wid * TOK_PER_W

    pltpu.sync_copy(iu_hbm.at[pl.ds(base, TOK_PER_W)], idx_u)
    pltpu.sync_copy(ij_hbm.at[pl.ds(base, TOK_PER_W)], idx_i)
    pltpu.sync_copy(ik_hbm.at[pl.ds(base, TOK_PER_W)], idx_k)

    bufs = ((rows_u0, rows_i0, rows_k0, sem0),
            (rows_u1, rows_i1, rows_k1, sem1))

    def issue(c):
        ru, ri, rk, sem = bufs[c % 2]
        off = c * CH
        return (
            pltpu.async_copy(uemb.at[idx_u.at[pl.ds(off, CH)]], ru, sem),
            pltpu.async_copy(iemb.at[idx_i.at[pl.ds(off, CH)]], ri, sem),
            pltpu.async_copy(temb.at[idx_k.at[pl.ds(off, CH)]], rk, sem),
        )

    pending = [None, None]
    pending[0] = issue(0)

    lanes = lax.iota(jnp.int32, 16)

    for c in range(NCH):
        if c + 1 < NCH:
            pending[(c + 1) % 2] = issue(c + 1)
        for cp in pending[c % 2]:
            cp.wait()
        rows_u, rows_i, rows_k, _ = bufs[c % 2]

        @plsc.parallel_loop(0, CH, unroll=4)
        def _tok(t):
            prods = []
            for s in range(D // 16):
                sl = pl.ds(s * 16, 16)
                prods.append(rows_u[t, sl] * rows_i[t, sl] * rows_k[t, sl])
            while len(prods) > 1:
                prods = [prods[i] + prods[i + 1]
                         for i in range(0, len(prods), 2)]
            rot = jnp.bitwise_and(lanes + t, 15)
            acc_v[pl.ds(t * 16, 16)] = jnp.take_along_axis(prods[0], rot, 0)

        for g in range(CH // 16):
            t_vec = g * 16 + lanes
            vals = []
            for l in range(16):
                idx = t_vec * 16 + jnp.bitwise_and(t_vec + l, 15)
                vals.append(plsc.load_gather(acc_v, [idx]))
            while len(vals) > 1:
                vals = [vals[i] + vals[i + 1] for i in range(0, len(vals), 2)]
            out_v[pl.ds(g * 16, 16)] = vals[0]

        pltpu.sync_copy(out_v, out_hbm.at[pl.ds(base + c * CH, CH)])


def kernel(i_input, j_input, k_input, user_embeddings, item_embeddings, time_embeddings):
    return _cp(
        i_input.astype(jnp.int32),
        j_input.astype(jnp.int32),
        k_input.astype(jnp.int32),
        user_embeddings,
        item_embeddings,
        time_embeddings,
    )


# speedup vs baseline: 1.0288x; 1.0288x over previous
import functools

import jax
import jax.numpy as jnp
from jax import lax
from jax.experimental import pallas as pl
from jax.experimental.pallas import tpu as pltpu
from jax.experimental.pallas import tpu_sc as plsc

B = 16384
D = 128
NC = 2
NS = 16
NW = NC * NS
TOK_PER_W = B // NW
CH = 128
NCH = TOK_PER_W // CH

_mesh = plsc.VectorSubcoreMesh(core_axis_name="c", subcore_axis_name="s")


@functools.partial(
    pl.kernel,
    out_type=jax.ShapeDtypeStruct((B,), jnp.float32),
    mesh=_mesh,
    compiler_params=pltpu.CompilerParams(needs_layout_passes=False),
    scratch_types=[
        pltpu.VMEM((TOK_PER_W,), jnp.int32),
        pltpu.VMEM((TOK_PER_W,), jnp.int32),
        pltpu.VMEM((TOK_PER_W,), jnp.int32),
        pltpu.VMEM((CH, D), jnp.float32),
        pltpu.VMEM((CH, D), jnp.float32),
        pltpu.VMEM((CH, D), jnp.float32),
        pltpu.VMEM((CH, D), jnp.float32),
        pltpu.VMEM((CH, D), jnp.float32),
        pltpu.VMEM((CH, D), jnp.float32),
        pltpu.VMEM((CH * 16,), jnp.float32),
        pltpu.VMEM((CH,), jnp.float32),
        pltpu.SemaphoreType.DMA,
        pltpu.SemaphoreType.DMA,
    ],
)
def _cp(iu_hbm, ij_hbm, ik_hbm, uemb, iemb, temb, out_hbm,
        idx_u, idx_i, idx_k,
        rows_u0, rows_i0, rows_k0, rows_u1, rows_i1, rows_k1,
        acc_v, out_v, sem0, sem1):
    wid = lax.axis_index("s") * NC + lax.axis_index("c")
    base = wid * TOK_PER_W

    pltpu.sync_copy(iu_hbm.at[pl.ds(base, TOK_PER_W)], idx_u)
    pltpu.sync_copy(ij_hbm.at[pl.ds(base, TOK_PER_W)], idx_i)
    pltpu.sync_copy(ik_hbm.at[pl.ds(base, TOK_PER_W)], idx_k)

    bufs = ((rows_u0, rows_i0, rows_k0, sem0),
            (rows_u1, rows_i1, rows_k1, sem1))

    def issue(c):
        ru, ri, rk, sem = bufs[c % 2]
        off = c * CH
        return (
            pltpu.async_copy(uemb.at[idx_u.at[pl.ds(off, CH)]], ru, sem),
            pltpu.async_copy(iemb.at[idx_i.at[pl.ds(off, CH)]], ri, sem),
            pltpu.async_copy(temb.at[idx_k.at[pl.ds(off, CH)]], rk, sem),
        )

    pending = [None, None]
    pending[0] = issue(0)

    lanes = lax.iota(jnp.int32, 16)

    for c in range(NCH):
        if c + 1 < NCH:
            pending[(c + 1) % 2] = issue(c + 1)
        for cp in pending[c % 2]:
            cp.wait()
        rows_u, rows_i, rows_k, _ = bufs[c % 2]

        @plsc.parallel_loop(0, CH, unroll=8)
        def _tok(t):
            prods = []
            for s in range(D // 16):
                sl = pl.ds(s * 16, 16)
                prods.append(rows_u[t, sl] * rows_i[t, sl] * rows_k[t, sl])
            while len(prods) > 1:
                prods = [prods[i] + prods[i + 1]
                         for i in range(0, len(prods), 2)]
            acc_v[pl.ds(t * 16, 16)] = prods[0]

        for g in range(CH // 16):
            flat = (g * 16 + lanes) * 16
            vals = [plsc.load_gather(acc_v, [flat + l]) for l in range(16)]
            while len(vals) > 1:
                vals = [vals[i] + vals[i + 1] for i in range(0, len(vals), 2)]
            out_v[pl.ds(g * 16, 16)] = vals[0]

        pltpu.sync_copy(out_v, out_hbm.at[pl.ds(base + c * CH, CH)])


def kernel(i_input, j_input, k_input, user_embeddings, item_embeddings, time_embeddings):
    return _cp(
        i_input.astype(jnp.int32),
        j_input.astype(jnp.int32),
        k_input.astype(jnp.int32),
        user_embeddings,
        item_embeddings,
        time_embeddings,
    )


# speedup vs baseline: 1.1323x; 1.1006x over previous
import functools

import jax
import jax.numpy as jnp
from jax import lax
from jax.experimental import pallas as pl
from jax.experimental.pallas import tpu as pltpu
from jax.experimental.pallas import tpu_sc as plsc

B = 16384
D = 128
NC = 2
NS = 16
NW = NC * NS
TOK_PER_W = B // NW
CH = 128
NCH = TOK_PER_W // CH

_mesh = plsc.VectorSubcoreMesh(core_axis_name="c", subcore_axis_name="s")


@functools.partial(
    pl.kernel,
    out_type=jax.ShapeDtypeStruct((B,), jnp.float32),
    mesh=_mesh,
    compiler_params=pltpu.CompilerParams(needs_layout_passes=False),
    scratch_types=[
        pltpu.VMEM((TOK_PER_W,), jnp.int32),
        pltpu.VMEM((TOK_PER_W,), jnp.int32),
        pltpu.VMEM((TOK_PER_W,), jnp.int32),
        pltpu.VMEM((CH, D), jnp.float32),
        pltpu.VMEM((CH, D), jnp.float32),
        pltpu.VMEM((CH, D), jnp.float32),
        pltpu.VMEM((CH, D), jnp.float32),
        pltpu.VMEM((CH, D), jnp.float32),
        pltpu.VMEM((CH, D), jnp.float32),
        pltpu.VMEM((CH * 16,), jnp.float32),
        pltpu.VMEM((CH,), jnp.float32),
        pltpu.SemaphoreType.DMA,
        pltpu.SemaphoreType.DMA,
    ],
)
def _cp(iu_hbm, ij_hbm, ik_hbm, uemb, iemb, temb, out_hbm,
        idx_u, idx_i, idx_k,
        rows_u0, rows_i0, rows_k0, rows_u1, rows_i1, rows_k1,
        acc_v, out_v, sem0, sem1):
    wid = lax.axis_index("s") * NC + lax.axis_index("c")
    base = wid * TOK_PER_W

    pltpu.sync_copy(iu_hbm.at[pl.ds(base, TOK_PER_W)], idx_u)
    pltpu.sync_copy(ij_hbm.at[pl.ds(base, TOK_PER_W)], idx_i)
    pltpu.sync_copy(ik_hbm.at[pl.ds(base, TOK_PER_W)], idx_k)

    bufs = ((rows_u0, rows_i0, rows_k0, sem0),
            (rows_u1, rows_i1, rows_k1, sem1))

    def issue(c):
        ru, ri, rk, sem = bufs[c % 2]
        off = c * CH
        return (
            pltpu.async_copy(uemb.at[idx_u.at[pl.ds(off, CH)]], ru, sem),
            pltpu.async_copy(iemb.at[idx_i.at[pl.ds(off, CH)]], ri, sem),
            pltpu.async_copy(temb.at[idx_k.at[pl.ds(off, CH)]], rk, sem),
        )

    pending = [None, None]
    pending[0] = issue(0)

    lanes = lax.iota(jnp.int32, 16)

    for c in range(NCH):
        if c + 1 < NCH:
            pending[(c + 1) % 2] = issue(c + 1)
        for cp in pending[c % 2]:
            cp.wait()
        rows_u, rows_i, rows_k, _ = bufs[c % 2]

        @plsc.parallel_loop(0, CH, unroll=4)
        def _tok(t):
            prods = []
            for s in range(D // 16):
                sl = pl.ds(s * 16, 16)
                prods.append(rows_u[t, sl] * rows_i[t, sl] * rows_k[t, sl])
            while len(prods) > 1:
                prods = [prods[i] + prods[i + 1]
                         for i in range(0, len(prods), 2)]
            acc_v[pl.ds(t * 16, 16)] = prods[0]

        for g in range(CH // 16):
            flat = (g * 16 + lanes) * 16
            vals = [plsc.load_gather(acc_v, [flat + l]) for l in range(16)]
            while len(vals) > 1:
                vals = [vals[i] + vals[i + 1] for i in range(0, len(vals), 2)]
            out_v[pl.ds(g * 16, 16)] = vals[0]

        pltpu.sync_copy(out_v, out_hbm.at[pl.ds(base + c * CH, CH)])


def kernel(i_input, j_input, k_input, user_embeddings, item_embeddings, time_embeddings):
    return _cp(
        i_input.astype(jnp.int32),
        j_input.astype(jnp.int32),
        k_input.astype(jnp.int32),
        user_embeddings,
        item_embeddings,
        time_embeddings,
    )
